# baseline (speedup 1.0000x reference)
import functools

import jax
import jax.numpy as jnp
from jax.experimental import pallas as pl
from jax.experimental.pallas import tpu as pltpu

_HID = 192
_N_SIZES = [10000, 20000, 40000]
_N_PADS = [10240, 20480, 40960]
_BN = 512


def _mm_body(x_ref, w_ref, o_ref):
    o_ref[...] = jnp.dot(x_ref[...], w_ref[...],
                         preferred_element_type=jnp.float32,
                         precision=jax.lax.Precision.DEFAULT)


def _matmul(x, w):
    n_pad, d_pad = x.shape
    f = w.shape[1]
    grid = (n_pad // _BN,)
    return pl.pallas_call(
        _mm_body,
        grid=grid,
        in_specs=[
            pl.BlockSpec((_BN, d_pad), lambda i: (i, 0)),
            pl.BlockSpec((d_pad, f), lambda i: (0, 0)),
        ],
        out_specs=pl.BlockSpec((_BN, f), lambda i: (i, 0)),
        out_shape=jax.ShapeDtypeStruct((n_pad, f), jnp.float32),
    )(x, w)


def _ep_body_plain(xw0_ref, agg_ref, b_ref, o_ref, *, act):
    t = xw0_ref[...] + agg_ref[...] + b_ref[...]
    if act:
        t = jnp.maximum(t, 0.0)
    o_ref[...] = t


def _ep_body_res(xw0_ref, agg_ref, b_ref, res_ref, o_ref):
    t = jnp.maximum(xw0_ref[...] + agg_ref[...] + b_ref[...], 0.0)
    o_ref[...] = 0.5 * (res_ref[...] + t)


def _epilogue(xw0, agg, b, res=None, act=True):
    n_pad, f = xw0.shape
    grid = (n_pad // _BN,)
    row = pl.BlockSpec((_BN, f), lambda i: (i, 0))
    brow = pl.BlockSpec((1, f), lambda i: (0, 0))
    if res is None:
        return pl.pallas_call(
            functools.partial(_ep_body_plain, act=act),
            grid=grid,
            in_specs=[row, row, brow],
            out_specs=row,
            out_shape=jax.ShapeDtypeStruct((n_pad, f), jnp.float32),
        )(xw0, agg, b)
    return pl.pallas_call(
        _ep_body_res,
        grid=grid,
        in_specs=[row, row, brow, row],
        out_specs=row,
        out_shape=jax.ShapeDtypeStruct((n_pad, f), jnp.float32),
    )(xw0, agg, b, res)


def _round_up(v, m):
    return (v + m - 1) // m * m


def _graph_conv(x_pad, p, src, dst, n, act=True, res=None):
    n_pad, d_pad = x_pad.shape
    dout = p["W0"].shape[1]
    dout_p = max(16, _round_up(dout, 16))
    w0 = jnp.pad(p["W0"], ((0, d_pad - p["W0"].shape[0]), (0, dout_p - dout)))
    w1 = jnp.pad(p["W1"], ((0, d_pad - p["W1"].shape[0]), (0, dout_p - dout)))
    wcat = jnp.concatenate([w0, w1], axis=1)
    y = _matmul(x_pad, wcat)
    xw0 = y[:, :dout_p]
    xw1 = y[:, dout_p:]
    agg = jax.ops.segment_sum(xw1[src], dst, num_segments=n_pad)
    b = jnp.pad(p["b"], (0, dout_p - dout))[None, :]
    return _epilogue(xw0, agg, b, res=res, act=act)


def _conv(x, w, b, stride=1):
    y = jax.lax.conv_general_dilated(x, w, (stride, stride), "SAME",
                                     dimension_numbers=("NHWC", "HWIO", "NHWC"))
    return jax.nn.relu(y + b)


def _cnn18(img, cnn):
    x = img[None]
    feats = []
    for i in range(6):
        p = cnn[i]
        x = _conv(x, p["c1W"], p["c1b"])
        x = _conv(x, p["c2W"], p["c2b"])
        if i >= 2:
            feats.append(x[0])
        x = _conv(x, p["sW"], p["sb"], 2)
    return feats


def _bilinear(feat, u, v):
    s = feat.shape[0]
    u0 = jnp.clip(jnp.floor(u).astype(jnp.int32), 0, s - 1)
    v0 = jnp.clip(jnp.floor(v).astype(jnp.int32), 0, s - 1)
    u1 = jnp.clip(u0 + 1, 0, s - 1)
    v1 = jnp.clip(v0 + 1, 0, s - 1)
    du = (u - u0.astype(u.dtype))[:, None]
    dv = (v - v0.astype(v.dtype))[:, None]
    f00 = feat[v0, u0]
    f01 = feat[v0, u1]
    f10 = feat[v1, u0]
    f11 = feat[v1, u1]
    return (f00 * (1 - du) * (1 - dv) + f01 * du * (1 - dv)
            + f10 * (1 - du) * dv + f11 * du * dv)


def _projection(x, img_feats):
    xc, yc = x[:, 0], x[:, 1]
    parts = [x]
    for feat in img_feats:
        s = feat.shape[0]
        u = (jnp.tanh(xc) * 0.5 + 0.5) * (s - 1)
        v = (jnp.tanh(yc) * 0.5 + 0.5) * (s - 1)
        parts.append(_bilinear(feat, u, v))
    return jnp.concatenate(parts, axis=1)


def _unpool(x, idx):
    new = 0.5 * (x[idx[:, 0]] + x[idx[:, 1]])
    return jnp.concatenate([x, new], axis=0)


def kernel(img_input, features, edge_index0, edge_index1, edge_index2,
           pool_idx0, pool_idx1, params):
    eis = [edge_index0, edge_index1, edge_index2]
    pis = [pool_idx0, pool_idx1]
    img_feats = _cnn18(img_input, params["cnn"])
    x = features
    outputs, outputs_unpool = [], []
    x_conv = None
    for i in range(3):
        n, n_pad = _N_SIZES[i], _N_PADS[i]
        x_proj = _projection(x, img_feats)
        if i > 0:
            outputs_unpool.append(_unpool(x, pis[i - 1]))
            x_proj = jnp.concatenate([x_proj, x_conv], axis=1)
            x_proj = _unpool(x_proj, pis[i - 1])
        d = x_proj.shape[1]
        d_pad = _round_up(d, 128)
        xp = jnp.pad(x_proj, ((0, n_pad - n), (0, d_pad - d)))
        src, dst = eis[i][0], eis[i][1]
        st = params["gcn"][i]
        h = _graph_conv(xp, st["gc_in"], src, dst, n, act=True)
        for rb in st["res"]:
            h1 = _graph_conv(h, rb["gc1"], src, dst, n, act=True)
            h = _graph_conv(h1, rb["gc2"], src, dst, n, act=True, res=h)
        x_conv = h[:n]
        if i == 2:
            y = _graph_conv(h, st["final"][0], src, dst, n, act=True)
            yp = jnp.pad(y, ((0, 0), (0, 128 - y.shape[1])))
            xo = _graph_conv(yp, st["final"][1], src, dst, n, act=False)
        else:
            xo = _graph_conv(h, st["final"][0], src, dst, n, act=False)
        x = xo[:n, :3]
        outputs.append(x)
    return tuple(outputs) + tuple(outputs_unpool)
